# baseline (speedup 1.0000x reference)
import functools

import jax
import jax.numpy as jnp
from jax import lax
from jax.experimental import pallas as pl
from jax.experimental.pallas import tpu as pltpu
from jax.experimental.pallas import tpu_sc as plsc

D = 64
WINDOW = 8
LANES = 16
BS = 128
NBUF = 4
OBUF = 2


def kernel(input, table, b_embed):
    B, N = input.shape
    BN = B * N
    idx_t = input.T.reshape(BN).astype(jnp.int32)

    info = plsc.get_sparse_core_info()
    num_workers = info.num_cores * info.num_subcores
    n_bt = B // BS
    total_chunks = N * n_bt
    per_w = total_chunks // num_workers
    idx_per_w = per_w * BS
    assert per_w * num_workers == total_chunks and per_w % NBUF == 0

    @functools.partial(
        pl.kernel,
        out_type=jax.ShapeDtypeStruct((N, D // 8, n_bt, 8, BS), jnp.float32),
        mesh=plsc.VectorSubcoreMesh(core_axis_name="c", subcore_axis_name="s"),
        compiler_params=pltpu.CompilerParams(
            use_tc_tiling_on_sc=False, needs_layout_passes=False
        ),
        scratch_types=[
            pltpu.VMEM((idx_per_w,), jnp.int32),
            pltpu.VMEM((NBUF, BS, D), jnp.float32),
            pltpu.VMEM((OBUF, D // 8, 8, BS), jnp.float32),
            pltpu.VMEM((D,), jnp.float32),
        ]
        + [pltpu.SemaphoreType.DMA] * (NBUF + OBUF),
    )
    def body(idx_hbm, table_hbm, bias_hbm, out_hbm, idx_all, rows, obuf, b_v, *sems):
        gsem = sems[:NBUF]
        osem = sems[NBUF:]
        wid = lax.axis_index("s") * info.num_cores + lax.axis_index("c")
        base_c = wid * per_w
        pltpu.sync_copy(bias_hbm, b_v)
        pltpu.sync_copy(idx_hbm.at[pl.ds(base_c * BS, idx_per_w)], idx_all)

        iota16 = lax.iota(jnp.int32, LANES)

        def gather_start(c, p):
            src = table_hbm.at[idx_all.at[pl.ds(c * BS, BS)]]
            pltpu.async_copy(src, rows.at[p], gsem[p])

        def gather_wait(p):
            src = table_hbm.at[idx_all.at[pl.ds(0, BS)]]
            pltpu.make_async_copy(src, rows.at[p], gsem[p]).wait()

        def store_start(n, bt, q):
            pltpu.async_copy(obuf.at[q], out_hbm.at[n, :, bt], osem[q])

        def store_wait(q):
            pltpu.make_async_copy(obuf.at[q], out_hbm.at[0, :, 0], osem[q]).wait()

        def add_bias(p):
            def one_row(r, _):
                for k in range(D // LANES):
                    sl = pl.ds(k * LANES, LANES)
                    rows[p, r, sl] = rows[p, r, sl] + b_v[sl]
                return 0

            lax.fori_loop(0, BS, one_row, 0)

        def transpose_chunk(p, q):
            def one_dt(dt, _):
                for ds_ in range(8):
                    col = jnp.full((LANES,), dt * 8 + ds_, jnp.int32)
                    for g in range(8):
                        bs_vec = g * LANES + iota16
                        v = plsc.load_gather(rows.at[p], [bs_vec, col])
                        obuf[q, dt, ds_, pl.ds(g * LANES, LANES)] = v
                return 0

            lax.fori_loop(0, D // 8, one_dt, 0)

        for p in range(NBUF):
            gather_start(p, p)

        def outer(t, _):
            for p in range(NBUF):
                c = t * NBUF + p
                c_id = base_c + c
                n = c_id // n_bt
                bt = c_id % n_bt
                q = p % OBUF
                gather_wait(p)

                @pl.when(n % WINDOW == 0)
                def _():
                    add_bias(p)

                @pl.when(c >= OBUF)
                def _():
                    store_wait(q)

                transpose_chunk(p, q)
                store_start(n, bt, q)

                @pl.when(c + NBUF < per_w)
                def _():
                    gather_start(c + NBUF, p)

            return 0

        lax.fori_loop(0, per_w // NBUF, outer, 0)
        for q in range(OBUF):
            store_wait(q)

    out5 = body(idx_t, table, b_embed)
    r = jnp.transpose(out5, (0, 1, 3, 2, 4)).reshape(N, D, B)
    return jnp.transpose(r, (2, 0, 1))
